# Initial kernel scaffold; baseline (speedup 1.0000x reference)
import functools

import jax
import jax.numpy as jnp
from jax import lax
from jax.experimental import pallas as pl
from jax.experimental.pallas import tpu as pltpu
from jax.experimental.pallas import tpu_sc as plsc

D = 64
B = 16384 * 50
NC, NS = 2, 16
NW = NC * NS
BPW = B // NW
CHUNK = 512
SUB = CHUNK // 128
NCHUNKS = BPW // CHUNK


def _make_gather():
    mesh = plsc.VectorSubcoreMesh(core_axis_name="c", subcore_axis_name="s")

    @functools.partial(
        pl.kernel,
        mesh=mesh,
        out_type=jax.ShapeDtypeStruct((B, D), jnp.float32),
        scratch_types=[
            pltpu.VMEM((SUB, 128), jnp.int32),
            pltpu.VMEM((CHUNK, D), jnp.float32),
            pltpu.SemaphoreType.DMA,
        ],
    )
    def gather_kernel(x_hbm, table_hbm, out_hbm, idx_v, rows_v, sem):
        wid = lax.axis_index("s") * NC + lax.axis_index("c")

        def body(c, carry):
            r0 = wid * (BPW // 128) + c * SUB
            off = wid * BPW + c * CHUNK
            pltpu.sync_copy(x_hbm.at[pl.ds(r0, SUB)], idx_v)
            copies = [
                pltpu.async_copy(
                    table_hbm.at[idx_v.at[j]],
                    rows_v.at[pl.ds(j * 128, 128)],
                    sem,
                )
                for j in range(SUB)
            ]
            for cp in copies:
                cp.wait()
            pltpu.sync_copy(rows_v, out_hbm.at[pl.ds(off, CHUNK)])
            return carry

        lax.fori_loop(0, NCHUNKS, body, 0)

    return gather_kernel


_gather = _make_gather()


def kernel(x, embedding):
    xf = x.reshape(-1).astype(jnp.int32).reshape(B // 128, 128)
    out = _gather(xf, embedding)
    return out.reshape(x.shape[0], x.shape[1], D)


# speedup vs baseline: 1.7951x; 1.7951x over previous
import functools

import jax
import jax.numpy as jnp
from jax import lax
from jax.experimental import pallas as pl
from jax.experimental.pallas import tpu as pltpu
from jax.experimental.pallas import tpu_sc as plsc

D = 64
B = 16384 * 50
NC, NS = 2, 16
NW = NC * NS
BPW = B // NW
CHUNK = 512
SUB = CHUNK // 128
NCHUNKS = BPW // CHUNK


def _make_gather():
    mesh = plsc.VectorSubcoreMesh(core_axis_name="c", subcore_axis_name="s")

    @functools.partial(
        pl.kernel,
        mesh=mesh,
        out_type=jax.ShapeDtypeStruct((B, D), jnp.float32),
        scratch_types=[
            pltpu.VMEM((SUB, 128), jnp.int32),
            pltpu.VMEM((CHUNK, D), jnp.float32),
            pltpu.SemaphoreType.DMA,
        ],
        compiler_params=pltpu.CompilerParams(use_tc_tiling_on_sc=False),
    )
    def gather_kernel(x_hbm, table_hbm, out_hbm, idx_v, rows_v, sem):
        wid = lax.axis_index("s") * NC + lax.axis_index("c")

        def body(c, carry):
            r0 = wid * (BPW // 128) + c * SUB
            off = wid * BPW + c * CHUNK
            pltpu.sync_copy(x_hbm.at[pl.ds(r0, SUB)], idx_v)
            copies = [
                pltpu.async_copy(
                    table_hbm.at[idx_v.at[j]],
                    rows_v.at[pl.ds(j * 128, 128)],
                    sem,
                )
                for j in range(SUB)
            ]
            for cp in copies:
                cp.wait()
            pltpu.sync_copy(rows_v, out_hbm.at[pl.ds(off, CHUNK)])
            return carry

        lax.fori_loop(0, NCHUNKS, body, 0)

    return gather_kernel


_gather = _make_gather()


def kernel(x, embedding):
    xf = x.reshape(-1).astype(jnp.int32).reshape(B // 128, 128)
    out = _gather(xf, embedding)
    return out.reshape(x.shape[0], x.shape[1], D)


# speedup vs baseline: 1.8426x; 1.0264x over previous
import functools

import jax
import jax.numpy as jnp
from jax import lax
from jax.experimental import pallas as pl
from jax.experimental.pallas import tpu as pltpu
from jax.experimental.pallas import tpu_sc as plsc

D = 64
B = 16384 * 50
NC, NS = 2, 16
NW = NC * NS
BPW = B // NW
CHUNK = 512
SUB = CHUNK // 128
NBUF = 2
NCHUNKS = BPW // CHUNK
NGROUPS = NCHUNKS // NBUF


def _make_gather():
    mesh = plsc.VectorSubcoreMesh(core_axis_name="c", subcore_axis_name="s")

    @functools.partial(
        pl.kernel,
        mesh=mesh,
        out_type=jax.ShapeDtypeStruct((B, D), jnp.float32),
        scratch_types=[
            [pltpu.VMEM((SUB, 128), jnp.int32) for _ in range(NBUF)],
            [pltpu.VMEM((CHUNK, D), jnp.float32) for _ in range(NBUF)],
            [pltpu.SemaphoreType.DMA for _ in range(NBUF)],
            [pltpu.SemaphoreType.DMA for _ in range(NBUF)],
            [pltpu.SemaphoreType.DMA for _ in range(NBUF)],
        ],
        compiler_params=pltpu.CompilerParams(use_tc_tiling_on_sc=False),
    )
    def gather_kernel(x_hbm, table_hbm, out_hbm, idx_v, rows_v, isem, gsem, osem):
        wid = lax.axis_index("s") * NC + lax.axis_index("c")

        def body(g, carry):
            icopies = []
            for b in range(NBUF):
                c = g * NBUF + b
                r0 = wid * (BPW // 128) + c * SUB
                icopies.append(
                    pltpu.async_copy(x_hbm.at[pl.ds(r0, SUB)], idx_v[b], isem[b])
                )
            gcopies = []
            for b in range(NBUF):
                icopies[b].wait()
                gcopies.append([
                    pltpu.async_copy(
                        table_hbm.at[idx_v[b].at[j]],
                        rows_v[b].at[pl.ds(j * 128, 128)],
                        gsem[b],
                    )
                    for j in range(SUB)
                ])
            ocopies = []
            for b in range(NBUF):
                c = g * NBUF + b
                off = wid * BPW + c * CHUNK
                for cp in gcopies[b]:
                    cp.wait()
                ocopies.append(
                    pltpu.async_copy(rows_v[b], out_hbm.at[pl.ds(off, CHUNK)], osem[b])
                )
            for cp in ocopies:
                cp.wait()
            return carry

        lax.fori_loop(0, NGROUPS, body, 0)

    return gather_kernel


_gather = _make_gather()


def kernel(x, embedding):
    xf = x.reshape(-1).astype(jnp.int32).reshape(B // 128, 128)
    out = _gather(xf, embedding)
    return out.reshape(x.shape[0], x.shape[1], D)
